# Initial kernel scaffold; baseline (speedup 1.0000x reference)
import jax
import jax.numpy as jnp
from jax.experimental import pallas as pl


def kernel(h, edge_index, edge_attr, batch, W0, b0, g0, be0, W1, b1, g1, be1, W2, b2, g2, be2, W3, b3, g3, be3, fc1W, fc1b, fc2W, fc2b):
    raise NotImplementedError("write your pallas kernel here")


# speedup vs baseline: 5.5918x; 5.5918x over previous
import jax
import jax.numpy as jnp
from jax import lax
from jax.experimental import pallas as pl
from jax.experimental.pallas import tpu as pltpu
from jax.experimental.pallas import tpu_sc as plsc

_N, _E, _D, _G, _C = 10000, 320000, 128, 64, 10
_BN_INV = (1.0 + 1e-5) ** -0.5
_NC, _NS = 2, 16
_NW = _NC * _NS
_EK = 128
_NCHUNK = _E // _EK
_ZR = 400
_NZCH = _N // _ZR


def _edge_agg_body(x_hbm, src_hbm, dst_hbm, zeros_hbm, out_hbm,
                   acc, src_v, dst_v, rows_v, sem):
    cid = lax.axis_index("c")
    sid = lax.axis_index("s")
    w = sid * _NC + cid

    nz = (_NZCH - sid + _NS - 1) // _NS

    def zbody(j, c):
        r = pl.multiple_of((j * _NS + sid) * _ZR, 8)
        pltpu.sync_copy(zeros_hbm.at[pl.ds(r, _ZR)], acc.at[pl.ds(r, _ZR)])
        return c

    lax.fori_loop(0, nz, zbody, 0)
    plsc.subcore_barrier()

    n_chunks = (_NCHUNK - w + _NW - 1) // _NW

    def body(i, carry):
        base = pl.multiple_of((i * _NW + w) * _EK, _EK)
        pltpu.sync_copy(src_hbm.at[pl.ds(base, _EK)], src_v)
        pltpu.sync_copy(dst_hbm.at[pl.ds(base, _EK)], dst_v)
        pltpu.async_copy(x_hbm.at[src_v], rows_v, sem).wait()
        pltpu.sync_copy(rows_v, acc.at[dst_v], add=True)
        return carry

    lax.fori_loop(0, n_chunks, body, 0)
    plsc.subcore_barrier()

    def wbody(j, c):
        r = pl.multiple_of((j * _NS + sid) * _ZR, 8)
        pltpu.sync_copy(acc.at[pl.ds(r, _ZR)],
                        out_hbm.at[cid, pl.ds(r, _ZR)])
        return c

    lax.fori_loop(0, nz, wbody, 0)


_edge_agg_cache = []


def _edge_agg(x, src, dst, zeros):
    if not _edge_agg_cache:
        _edge_agg_cache.append(pl.kernel(
            _edge_agg_body,
            out_type=jax.ShapeDtypeStruct((_NC, _N, _D), jnp.float32),
            mesh=plsc.VectorSubcoreMesh(core_axis_name="c",
                                        subcore_axis_name="s",
                                        num_cores=_NC, num_subcores=_NS),
            scratch_types=[
                pltpu.VMEM_SHARED((_N, _D), jnp.float32),
                pltpu.VMEM((_EK,), jnp.int32),
                pltpu.VMEM((_EK,), jnp.int32),
                pltpu.VMEM((_EK, _D), jnp.float32),
                pltpu.SemaphoreType.DMA,
            ],
        ))
    return _edge_agg_cache[0](x, src, dst, zeros)


def _layer_body(x_ref, p_ref, w_ref, b_ref, g_ref, be_ref, o_ref):
    z = x_ref[...] + (p_ref[0] + p_ref[1])
    z = jnp.dot(z, w_ref[...], preferred_element_type=jnp.float32)
    z = z + b_ref[...]
    z = (z / jnp.sqrt(1.0 + 1e-5)) * g_ref[...] + be_ref[...]
    o_ref[...] = jnp.maximum(z, 0.0)


_ROW_BLK = 1000


def _layer(x, p, W, b, g, be):
    return pl.pallas_call(
        _layer_body,
        grid=(_N // _ROW_BLK,),
        in_specs=[
            pl.BlockSpec((_ROW_BLK, _D), lambda i: (i, 0)),
            pl.BlockSpec((_NC, _ROW_BLK, _D), lambda i: (0, i, 0)),
            pl.BlockSpec((_D, _D), lambda i: (0, 0)),
            pl.BlockSpec((1, _D), lambda i: (0, 0)),
            pl.BlockSpec((1, _D), lambda i: (0, 0)),
            pl.BlockSpec((1, _D), lambda i: (0, 0)),
        ],
        out_specs=pl.BlockSpec((_ROW_BLK, _D), lambda i: (i, 0)),
        out_shape=jax.ShapeDtypeStruct((_N, _D), jnp.float32),
    )(x, p, W, b.reshape(1, _D), g.reshape(1, _D), be.reshape(1, _D))


def _final_body(x_ref, batch_ref, f1w_ref, f1b_ref, f2w_ref, f2b_ref, o_ref):
    gid = lax.broadcasted_iota(jnp.int32, (_G, _N), 0)
    onehot = (batch_ref[...] == gid).astype(jnp.float32)
    cnt = jnp.sum(onehot, axis=1, keepdims=True)
    sums = jnp.dot(onehot, x_ref[...], preferred_element_type=jnp.float32,
                   precision=lax.Precision.HIGHEST)
    pooled = sums / jnp.maximum(cnt, 1.0)
    z = jnp.dot(pooled, f1w_ref[...], preferred_element_type=jnp.float32)
    z = z + f1b_ref[...]
    z = jnp.where(z > 0.0, z, jnp.exp(jnp.minimum(z, 0.0)) - 1.0)
    z = jnp.dot(z, f2w_ref[...], preferred_element_type=jnp.float32)
    z = z + f2b_ref[...]
    m = jnp.max(z, axis=0, keepdims=True)
    lse = jnp.log(jnp.sum(jnp.exp(z - m), axis=0, keepdims=True))
    o_ref[...] = z - m - lse


def _final(x, batch, fc1W, fc1b, fc2W, fc2b):
    return pl.pallas_call(
        _final_body,
        out_shape=jax.ShapeDtypeStruct((_G, _C), jnp.float32),
    )(x, batch.reshape(1, _N), fc1W, fc1b.reshape(1, _D),
      fc2W, fc2b.reshape(1, _C))


def kernel(h, edge_index, edge_attr, batch,
           W0, b0, g0, be0, W1, b1, g1, be1,
           W2, b2, g2, be2, W3, b3, g3, be3,
           fc1W, fc1b, fc2W, fc2b):
    del edge_attr
    zeros = jnp.zeros((_N, _D), jnp.float32)
    src = edge_index[0]
    dst = edge_index[1]
    x = h
    for (W, b, g, be) in ((W0, b0, g0, be0), (W1, b1, g1, be1),
                          (W2, b2, g2, be2), (W3, b3, g3, be3)):
        p = _edge_agg(x, src, dst, zeros)
        x = _layer(x, p, W, b, g, be)
    return _final(x, batch, fc1W, fc1b, fc2W, fc2b)
